# baseline (speedup 1.0000x reference)
import functools

import jax
import jax.numpy as jnp
from jax import lax
from jax.experimental import pallas as pl
from jax.experimental.pallas import tpu as pltpu
from jax.experimental.pallas import tpu_sc as plsc

B = 4
S = 2048
IN_F = 1024
OUT_F = 1024
NUM_BANKS = 16

NW = 32
ROWS_PER_W = B * IN_F // NW
CH = 16
NCHUNK = ROWS_PER_W // CH
CHW = CH * OUT_F


def _sc_body(w2d, sel16, bw16, bias2d, weff1d, beff1d,
             buf0, buf1, obuf, bb, sel_vm, bw_vm, sem):
    wid = lax.axis_index("c") * 16 + lax.axis_index("s")
    b = wid // 8
    pltpu.sync_copy(sel16, sel_vm)
    pltpu.sync_copy(bw16, bw_vm)
    iota = lax.broadcasted_iota(jnp.int32, (16,), 0)
    lane0 = jnp.full((16,), 2 * b, jnp.int32)
    sel_v = sel_vm[...]
    bw_v = bw_vm[...]
    s0_spl = sel_v.at[lane0].get(mode="promise_in_bounds")
    s1_spl = sel_v.at[lane0 + 1].get(mode="promise_in_bounds")
    bw0 = bw_v.at[lane0].get(mode="promise_in_bounds")
    bw1 = bw_v.at[lane0 + 1].get(mode="promise_in_bounds")

    @pl.when(wid % 8 == 0)
    def _bias():
        bidx = jnp.where(iota == 1, s1_spl, s0_spl)
        pltpu.async_copy(bias2d.at[bidx], bb, sem).wait()

        def bbody(j, _):
            beff = bw0 * bb[0, pl.ds(j * 16, 16)] + bw1 * bb[1, pl.ds(j * 16, 16)]
            obuf[pl.ds(j * 16, 16)] = beff
            return 0

        lax.fori_loop(0, OUT_F // 16, bbody, 0)
        pltpu.sync_copy(obuf.at[pl.ds(0, OUT_F)],
                        beff1d.at[pl.ds(b * OUT_F, OUT_F)])

    rloc = (wid % 8) * ROWS_PER_W

    def chunk(c, _):
        rows = rloc + c * CH + iota
        pltpu.async_copy(w2d.at[s0_spl * IN_F + rows], buf0, sem).wait()
        pltpu.async_copy(w2d.at[s1_spl * IN_F + rows], buf1, sem).wait()
        for r in range(CH):
            def body(j, _):
                y = bw0 * buf0[r, pl.ds(j * 16, 16)] + bw1 * buf1[r, pl.ds(j * 16, 16)]
                obuf[pl.ds(r * OUT_F + j * 16, 16)] = y
                return 0

            lax.fori_loop(0, OUT_F // 16, body, 0, unroll=8)
        pltpu.sync_copy(
            obuf, weff1d.at[pl.ds((b * IN_F + rloc + c * CH) * OUT_F, CHW)])
        return 0

    lax.fori_loop(0, NCHUNK, chunk, 0)


def _combine_sc(W, bank_weights, bank_selections, bias):
    sel16 = jnp.zeros((16,), jnp.int32).at[:2 * B].set(
        bank_selections.reshape(-1))
    bw16 = jnp.zeros((16,), jnp.float32).at[:2 * B].set(
        bank_weights.reshape(-1))
    mesh = plsc.VectorSubcoreMesh(core_axis_name="c", subcore_axis_name="s")
    k = functools.partial(
        pl.kernel,
        out_type=(jax.ShapeDtypeStruct((B * IN_F * OUT_F,), jnp.float32),
                  jax.ShapeDtypeStruct((B * OUT_F,), jnp.float32)),
        mesh=mesh,
        scratch_types=[
            pltpu.VMEM((CH, OUT_F), jnp.float32),
            pltpu.VMEM((CH, OUT_F), jnp.float32),
            pltpu.VMEM((CHW,), jnp.float32),
            pltpu.VMEM((16, OUT_F), jnp.float32),
            pltpu.VMEM((16,), jnp.int32),
            pltpu.VMEM((16,), jnp.float32),
            pltpu.SemaphoreType.DMA,
        ],
    )(_sc_body)
    weff1d, beff1d = k(W.reshape(NUM_BANKS * IN_F, OUT_F), sel16, bw16,
                       bias, )
    return (weff1d.reshape(B, IN_F, OUT_F), beff1d.reshape(B, 1, OUT_F))


def _tc_body(x_ref, weff_ref, beff_ref, out_ref):
    acc = jnp.dot(x_ref[0].astype(jnp.bfloat16),
                  weff_ref[0].astype(jnp.bfloat16),
                  preferred_element_type=jnp.float32)
    out_ref[0] = acc + beff_ref[0]


def kernel(tensor, bank_weights, bank_selections, W, bias):
    weff, beff = _combine_sc(W, bank_weights, bank_selections, bias)
    return pl.pallas_call(
        _tc_body,
        grid=(B,),
        in_specs=[
            pl.BlockSpec((1, S, IN_F), lambda b: (b, 0, 0)),
            pl.BlockSpec((1, IN_F, OUT_F), lambda b: (b, 0, 0)),
            pl.BlockSpec((1, 1, OUT_F), lambda b: (b, 0, 0)),
        ],
        out_specs=pl.BlockSpec((1, S, OUT_F), lambda b: (b, 0, 0)),
        out_shape=jax.ShapeDtypeStruct((B, S, OUT_F), jnp.float32),
    )(tensor, weff, beff)


# speedup vs baseline: 3.4082x; 3.4082x over previous
import jax
import jax.numpy as jnp
from jax.experimental import pallas as pl
from jax.experimental.pallas import tpu as pltpu

B = 4
S = 2048
IN_F = 1024
OUT_F = 1024
NUM_BANKS = 16
NI = 2
SB = S // NI


def _body(sel_ref, bw_ref, x_ref, w0_ref, w1_ref, bias_ref, out_ref):
    b = pl.program_id(0)
    bw0 = bw_ref[b, 0]
    bw1 = bw_ref[b, 1]
    w_eff = (bw0 * w0_ref[0] + bw1 * w1_ref[0]).astype(jnp.bfloat16)
    acc = jnp.dot(x_ref[0].astype(jnp.bfloat16), w_eff,
                  preferred_element_type=jnp.float32)
    s0 = sel_ref[b, 0]
    s1 = sel_ref[b, 1]
    b_eff = bw0 * bias_ref[s0, :] + bw1 * bias_ref[s1, :]
    out_ref[0] = acc + b_eff[None, :]


def kernel(tensor, bank_weights, bank_selections, W, bias):
    grid_spec = pltpu.PrefetchScalarGridSpec(
        num_scalar_prefetch=2,
        grid=(B, NI),
        in_specs=[
            pl.BlockSpec((1, SB, IN_F), lambda b, i, sel, bw: (b, i, 0)),
            pl.BlockSpec((1, IN_F, OUT_F), lambda b, i, sel, bw: (sel[b, 0], 0, 0)),
            pl.BlockSpec((1, IN_F, OUT_F), lambda b, i, sel, bw: (sel[b, 1], 0, 0)),
            pl.BlockSpec((NUM_BANKS, OUT_F), lambda b, i, sel, bw: (0, 0)),
        ],
        out_specs=pl.BlockSpec((1, SB, OUT_F), lambda b, i, sel, bw: (b, i, 0)),
    )
    return pl.pallas_call(
        _body,
        grid_spec=grid_spec,
        out_shape=jax.ShapeDtypeStruct((B, S, OUT_F), jnp.float32),
    )(bank_selections, bank_weights, tensor, W, W, bias)


# speedup vs baseline: 3.9527x; 1.1598x over previous
import jax
import jax.numpy as jnp
from jax.experimental import pallas as pl
from jax.experimental.pallas import tpu as pltpu

B = 4
S = 2048
IN_F = 1024
OUT_F = 1024
NUM_BANKS = 16
SB = S // 2


def _body(sel_ref, bw_ref, xl_ref, xh_ref, w0_ref, w1_ref, bias_ref, out_ref):
    b = pl.program_id(0)
    bw0 = bw_ref[b, 0]
    bw1 = bw_ref[b, 1]
    w_eff = (bw0 * w0_ref[0] + bw1 * w1_ref[0]).astype(jnp.bfloat16)
    s0 = sel_ref[b, 0]
    s1 = sel_ref[b, 1]
    b_eff = (bw0 * bias_ref[s0, :] + bw1 * bias_ref[s1, :])[None, :]
    out_ref[0, :SB] = jnp.dot(xl_ref[0].astype(jnp.bfloat16), w_eff,
                              preferred_element_type=jnp.float32) + b_eff
    out_ref[0, SB:] = jnp.dot(xh_ref[0].astype(jnp.bfloat16), w_eff,
                              preferred_element_type=jnp.float32) + b_eff


def kernel(tensor, bank_weights, bank_selections, W, bias):
    grid_spec = pltpu.PrefetchScalarGridSpec(
        num_scalar_prefetch=2,
        grid=(B,),
        in_specs=[
            pl.BlockSpec((1, SB, IN_F), lambda b, sel, bw: (b, 0, 0)),
            pl.BlockSpec((1, SB, IN_F), lambda b, sel, bw: (b, 1, 0)),
            pl.BlockSpec((1, IN_F, OUT_F), lambda b, sel, bw: (sel[b, 0], 0, 0)),
            pl.BlockSpec((1, IN_F, OUT_F), lambda b, sel, bw: (sel[b, 1], 0, 0)),
            pl.BlockSpec((NUM_BANKS, OUT_F), lambda b, sel, bw: (0, 0)),
        ],
        out_specs=pl.BlockSpec((1, S, OUT_F), lambda b, sel, bw: (b, 0, 0)),
    )
    return pl.pallas_call(
        _body,
        grid_spec=grid_spec,
        out_shape=jax.ShapeDtypeStruct((B, S, OUT_F), jnp.float32),
    )(bank_selections, bank_weights, tensor, tensor, W, W, bias)
